# baseline (speedup 1.0000x reference)
import functools

import jax
import jax.numpy as jnp
from jax import lax
from jax.experimental import pallas as pl
from jax.experimental.pallas import tpu as pltpu
from jax.experimental.pallas import tpu_sc as plsc

BATCH = 4096
D = 1000
DP = 1024
L = 16
NC, NS = 2, 16
NW = NC * NS
BPW = BATCH // NW
C = 8
NCHUNK = BPW // C
NV = D // L
TAIL = D - NV * L

_GATHER_DNUMS = lax.GatherDimensionNumbers(
    offset_dims=(), collapsed_slice_dims=(0,), start_index_map=(0,))


def _permute(v, idx):
    return lax.gather(v, idx[:, None], _GATHER_DNUMS, (1,),
                      mode=lax.GatherScatterMode.PROMISE_IN_BOUNDS)


def _sc_body(a_hbm, w_hbm, u_hbm, i_hbm, j_hbm, oi_hbm, oj_hbm,
             u_v, i_v, j_v, a_v, wi_v, wj_v, oi_v, oj_v,
             sem_a, sem_w):
    wid = lax.axis_index("s") * NC + lax.axis_index("c")
    base = wid * BPW
    pltpu.sync_copy(u_hbm.at[pl.ds(base, BPW)], u_v.at[pl.ds(0, BPW)])
    pltpu.sync_copy(i_hbm.at[pl.ds(base, BPW)], i_v)
    pltpu.sync_copy(j_hbm.at[pl.ds(base, BPW)], j_v)

    lane = lax.iota(jnp.int32, L)
    tail_mask = lane >= (L - TAIL)
    zero = jnp.zeros((L,), jnp.float32)

    def chunk_fn(c, chunk_res):
        off = c * C
        parity = lax.rem(c, 2)
        cpw1 = pltpu.async_copy(w_hbm.at[i_v.at[pl.ds(off, C)]], wi_v, sem_w)
        cpw2 = pltpu.async_copy(w_hbm.at[j_v.at[pl.ds(off, C)]], wj_v, sem_w)
        uvec = u_v[pl.ds(off, L)]
        us = [uvec[r] for r in range(C)]
        rms = [lax.rem(u, 8) for u in us]
        row_cps = []
        for r in range(C):
            g8 = pl.multiple_of(us[r] - rms[r], 8)
            cp = pltpu.async_copy(a_hbm.at[pl.ds(g8, 8)], a_v.at[r], sem_a)
            row_cps.append(cp)
        for cp in row_cps:
            cp.wait()
        cpw1.wait()
        cpw2.wait()

        res_i, res_j = chunk_res
        for r in range(C):
            rm = rms[r]

            def k_fn(s, acc, r=r, rm=rm):
                ai, aj = acc
                for t in range(8):
                    av = a_v[r, rm, pl.ds(s * 128 + t * L, L)]
                    ai = ai + av * wi_v[r, s, pl.ds(t * L, L)]
                    aj = aj + av * wj_v[r, s, pl.ds(t * L, L)]
                return ai, aj

            ai, aj = lax.fori_loop(0, 7, k_fn, (zero, zero))
            for t in range(6):
                av = a_v[r, rm, pl.ds(896 + t * L, L)]
                ai = ai + av * wi_v[r, 7, pl.ds(t * L, L)]
                aj = aj + av * wj_v[r, 7, pl.ds(t * L, L)]
            av = a_v[r, rm, pl.ds(D - L, L)]
            ai = ai + jnp.where(tail_mask, av * wi_v[r, 7, pl.ds(88, L)], 0.0)
            aj = aj + jnp.where(tail_mask, av * wj_v[r, 7, pl.ds(88, L)], 0.0)
            for s in (8, 4, 2, 1):
                perm = lane ^ s
                ai = ai + _permute(ai, perm)
                aj = aj + _permute(aj, perm)
            sel = lane == (r + parity * C)
            res_i = jnp.where(sel, ai, res_i)
            res_j = jnp.where(sel, aj, res_j)

        @pl.when(parity == 1)
        def _store():
            st = (c - 1) * C
            oi_v[pl.ds(st, 2 * C)] = res_i
            oj_v[pl.ds(st, 2 * C)] = res_j

        keep = parity == 0
        return (jnp.where(keep, res_i, zero), jnp.where(keep, res_j, zero))

    lax.fori_loop(0, NCHUNK, chunk_fn, (zero, zero))
    pltpu.sync_copy(oi_v, oi_hbm.at[pl.ds(base, BPW)])
    pltpu.sync_copy(oj_v, oj_hbm.at[pl.ds(base, BPW)])


def kernel(A, W, user, item_i, item_j):
    user = user.astype(jnp.int32)
    item_i = item_i.astype(jnp.int32)
    item_j = item_j.astype(jnp.int32)
    w_pad = jnp.pad(W, ((0, 0), (0, DP - D))).reshape(D, 8, 128)
    mesh = plsc.VectorSubcoreMesh(core_axis_name="c", subcore_axis_name="s")
    f32 = jnp.float32
    run = pl.kernel(
        _sc_body,
        out_type=(jax.ShapeDtypeStruct((BATCH,), f32),
                  jax.ShapeDtypeStruct((BATCH,), f32)),
        mesh=mesh,
        scratch_types=[
            pltpu.VMEM((BPW + L - C,), jnp.int32),
            pltpu.VMEM((BPW,), jnp.int32),
            pltpu.VMEM((BPW,), jnp.int32),
            pltpu.VMEM((C, 8, D), f32),
            pltpu.VMEM((C, 8, 128), f32),
            pltpu.VMEM((C, 8, 128), f32),
            pltpu.VMEM((BPW,), f32),
            pltpu.VMEM((BPW,), f32),
            pltpu.SemaphoreType.DMA,
            pltpu.SemaphoreType.DMA,
        ],
    )
    return run(A, w_pad, user, item_i, item_j)
